# baseline (speedup 1.0000x reference)
import functools

import jax
import jax.numpy as jnp
from jax import lax
from jax.experimental import pallas as pl
from jax.experimental.pallas import tpu as pltpu
from jax.experimental.pallas import tpu_sc as plsc

N_AXES = 4
PER_AXIS = 32
TABLE_ROWS = 4096
NUM_WORKERS = 32
TOK_PER_WORKER = 1024
CHUNK_TOK = 256
N_CHUNKS = TOK_PER_WORKER // CHUNK_TOK
IDS_PER_ROW = 128
ROWS_PER_AXIS = CHUNK_TOK // IDS_PER_ROW
NBUF = 2


def _embed_body(ids_hbm, w_hbm, out_hbm, idx_v, rows_v, sems):
    wid = lax.axis_index("s") * 2 + lax.axis_index("c")
    seq = out_hbm.shape[2]
    wpb = seq // TOK_PER_WORKER
    b = wid // wpb
    s0 = (wid % wpb) * TOK_PER_WORKER

    def load_chunk(c, buf):
        blk0 = s0 // IDS_PER_ROW + c * ROWS_PER_AXIS
        for i in range(N_AXES):
            pltpu.sync_copy(ids_hbm.at[b, pl.ds(blk0, ROWS_PER_AXIS), i],
                            idx_v.at[buf, i])
        for i in range(N_AXES):
            for j in range(ROWS_PER_AXIS):
                pltpu.make_async_copy(
                    w_hbm.at[i].at[idx_v.at[buf, i, j]],
                    rows_v.at[buf, i, pl.ds(j * IDS_PER_ROW, IDS_PER_ROW)],
                    sems.at[buf],
                ).start()

    def drain_chunk(c, buf):
        for i in range(N_AXES):
            for j in range(ROWS_PER_AXIS):
                pltpu.make_async_copy(
                    w_hbm.at[i].at[idx_v.at[buf, i, j]],
                    rows_v.at[buf, i, pl.ds(j * IDS_PER_ROW, IDS_PER_ROW)],
                    sems.at[buf],
                ).wait()
        for i in range(N_AXES):
            pltpu.sync_copy(
                rows_v.at[buf, i],
                out_hbm.at[b, 0, pl.ds(s0 + c * CHUNK_TOK, CHUNK_TOK),
                           pl.ds(i * PER_AXIS, PER_AXIS)])

    load_chunk(0, 0)
    for c in range(N_CHUNKS):
        if c + 1 < N_CHUNKS:
            load_chunk(c + 1, (c + 1) % NBUF)
        drain_chunk(c, c % NBUF)


def kernel(ids, W0, W1, W2, W3):
    batch, seq, n_axes = ids.shape
    ids_b = ids.astype(jnp.int32).reshape(
        batch, seq // IDS_PER_ROW, IDS_PER_ROW, n_axes).transpose(0, 1, 3, 2)

    mesh = plsc.VectorSubcoreMesh(core_axis_name="c", subcore_axis_name="s")
    run = functools.partial(
        pl.kernel,
        out_type=jax.ShapeDtypeStruct((batch, 1, seq, N_AXES * PER_AXIS),
                                      jnp.float32),
        mesh=mesh,
        scratch_types=[
            pltpu.VMEM((NBUF, N_AXES, ROWS_PER_AXIS, IDS_PER_ROW), jnp.int32),
            pltpu.VMEM((NBUF, N_AXES, CHUNK_TOK, PER_AXIS), jnp.float32),
            pltpu.SemaphoreType.DMA((NBUF,)),
        ],
        compiler_params=pltpu.CompilerParams(
            use_tc_tiling_on_sc=False, needs_layout_passes=False),
    )(_embed_body)
    return run(ids_b, jnp.stack([W0, W1, W2, W3]))


# speedup vs baseline: 1.1163x; 1.1163x over previous
import functools

import jax
import jax.numpy as jnp
from jax import lax
from jax.experimental import pallas as pl
from jax.experimental.pallas import tpu as pltpu
from jax.experimental.pallas import tpu_sc as plsc

N_AXES = 4
PER_AXIS = 32
TABLE_ROWS = 4096
NUM_WORKERS = 32
TOK_PER_WORKER = 1024
CHUNK_TOK = 256
N_CHUNKS = TOK_PER_WORKER // CHUNK_TOK
IDS_PER_ROW = 128
ROWS_PER_AXIS = CHUNK_TOK // IDS_PER_ROW
NBUF = 3


def _embed_body(ids_hbm, w0_hbm, w1_hbm, w2_hbm, w3_hbm, out_hbm,
                idx_v, rows_v, sems):
    w_hbm = [w0_hbm, w1_hbm, w2_hbm, w3_hbm]
    wid = lax.axis_index("s") * 2 + lax.axis_index("c")
    seq = out_hbm.shape[2]
    wpb = seq // TOK_PER_WORKER
    b = wid // wpb
    s0 = (wid % wpb) * TOK_PER_WORKER

    def load_chunk(c, buf):
        blk0 = s0 // IDS_PER_ROW + c * ROWS_PER_AXIS
        for i in range(N_AXES):
            pltpu.sync_copy(ids_hbm.at[b, pl.ds(blk0, ROWS_PER_AXIS), i],
                            idx_v.at[buf, i])
        for i in range(N_AXES):
            for j in range(ROWS_PER_AXIS):
                pltpu.make_async_copy(
                    w_hbm[i].at[idx_v.at[buf, i, j]],
                    rows_v.at[buf, i, pl.ds(j * IDS_PER_ROW, IDS_PER_ROW)],
                    sems.at[buf],
                ).start()

    def drain_chunk(c, buf):
        for i in range(N_AXES):
            for j in range(ROWS_PER_AXIS):
                pltpu.make_async_copy(
                    w_hbm[i].at[idx_v.at[buf, i, j]],
                    rows_v.at[buf, i, pl.ds(j * IDS_PER_ROW, IDS_PER_ROW)],
                    sems.at[buf],
                ).wait()
        for i in range(N_AXES):
            pltpu.sync_copy(
                rows_v.at[buf, i],
                out_hbm.at[b, 0, pl.ds(s0 + c * CHUNK_TOK, CHUNK_TOK),
                           pl.ds(i * PER_AXIS, PER_AXIS)])

    load_chunk(0, 0)
    load_chunk(1, 1)
    for c in range(N_CHUNKS):
        if c + 2 < N_CHUNKS:
            load_chunk(c + 2, (c + 2) % NBUF)
        drain_chunk(c, c % NBUF)


def kernel(ids, W0, W1, W2, W3):
    batch, seq, n_axes = ids.shape
    ids_b = ids.astype(jnp.int32).reshape(
        batch, seq // IDS_PER_ROW, IDS_PER_ROW, n_axes).transpose(0, 1, 3, 2)

    mesh = plsc.VectorSubcoreMesh(core_axis_name="c", subcore_axis_name="s")
    run = functools.partial(
        pl.kernel,
        out_type=jax.ShapeDtypeStruct((batch, 1, seq, N_AXES * PER_AXIS),
                                      jnp.float32),
        mesh=mesh,
        scratch_types=[
            pltpu.VMEM((NBUF, N_AXES, ROWS_PER_AXIS, IDS_PER_ROW), jnp.int32),
            pltpu.VMEM((NBUF, N_AXES, CHUNK_TOK, PER_AXIS), jnp.float32),
            pltpu.SemaphoreType.DMA((NBUF,)),
        ],
        compiler_params=pltpu.CompilerParams(
            use_tc_tiling_on_sc=False, needs_layout_passes=False),
    )(_embed_body)
    return run(ids_b, W0, W1, W2, W3)
